# baseline (speedup 1.0000x reference)
import functools

import jax
import jax.numpy as jnp
from jax import lax
from jax.experimental import pallas as pl
from jax.experimental.pallas import tpu as pltpu
from jax.experimental.pallas import tpu_sc as plsc

_NUM_EMB = 1024
_DIM = 64
_CC = 0.25
_ROWS = 16384
_TILE = 1024
_GRID = _ROWS // _TILE


def _vq_tc_body(z_ref, e_ref, idx_ref, loss_ref, perp_ref, counts_ref,
                sse_ref):
    i = pl.program_id(0)
    zt = z_ref[0]
    e = e_ref[...]
    mmn = lax.dot_general(e, -2.0 * zt, (((1,), (0,)), ((), ())))
    z2 = jnp.sum(zt * zt, axis=0)
    e2 = jnp.sum(e * e, axis=1)
    s = (e2[:, None] + z2[None, :]) + mmn
    md = jnp.min(s, axis=0)
    rows = lax.broadcasted_iota(jnp.int32, (_NUM_EMB, _TILE), 0)
    idx = jnp.min(jnp.where(s == md[None, :], rows, _NUM_EMB), axis=0)
    idx_ref[...] = idx
    onehot = (rows == idx[None, :]).astype(jnp.float32)
    cb = lax.dot_general(onehot, jnp.ones((_TILE, 1), jnp.float32),
                         (((1,), (0,)), ((), ())))

    @pl.when(i == 0)
    def _():
        counts_ref[...] = cb
        sse_ref[0] = jnp.sum(md)

    @pl.when(i > 0)
    def _():
        counts_ref[...] = counts_ref[...] + cb
        sse_ref[0] = sse_ref[0] + jnp.sum(md)

    @pl.when(i == _GRID - 1)
    def _():
        loss_ref[0, 0] = (1.0 + _CC) * sse_ref[0] / (_ROWS * _DIM)
        p = counts_ref[...] * (1.0 / _ROWS)
        ent = jnp.sum(p * jnp.log(p + 1e-10))
        perp_ref[0, 0] = jnp.exp(-ent)


def _tc_call(zt3, emb):
    return pl.pallas_call(
        _vq_tc_body,
        grid=(_GRID,),
        in_specs=[
            pl.BlockSpec((1, _DIM, _TILE), lambda i: (i, 0, 0)),
            pl.BlockSpec((_NUM_EMB, _DIM), lambda i: (0, 0)),
        ],
        out_specs=[
            pl.BlockSpec((_TILE,), lambda i: (i,)),
            pl.BlockSpec(memory_space=pltpu.SMEM),
            pl.BlockSpec(memory_space=pltpu.SMEM),
        ],
        out_shape=[
            jax.ShapeDtypeStruct((_ROWS,), jnp.int32),
            jax.ShapeDtypeStruct((1, 1), jnp.float32),
            jax.ShapeDtypeStruct((1, 1), jnp.float32),
        ],
        scratch_shapes=[
            pltpu.VMEM((_NUM_EMB, 1), jnp.float32),
            pltpu.SMEM((1,), jnp.float32),
        ],
    )(zt3, emb)


_NC, _NS = 2, 16
_NW = _NC * _NS
_BPW = _ROWS // _NW
_NB = 16
_HPX = 512


@functools.cache
def _sc_gather_call():
    @functools.partial(
        pl.kernel,
        mesh=plsc.VectorSubcoreMesh(core_axis_name="c", subcore_axis_name="s"),
        compiler_params=pltpu.CompilerParams(needs_layout_passes=False),
        out_type=jax.ShapeDtypeStruct((_NB, _DIM, 1024), jnp.float32),
        scratch_types=[
            pltpu.VMEM((_BPW,), jnp.int32),
            pltpu.VMEM((_NUM_EMB * _DIM,), jnp.float32),
            pltpu.VMEM((_DIM, _HPX), jnp.float32),
        ],
    )
    def _sc_gather(table_hbm, idx_hbm, out_hbm, idx_v, table_v, stage_v):
        wid = lax.axis_index("s") * _NC + lax.axis_index("c")
        base = wid * _BPW
        pltpu.sync_copy(table_hbm, table_v)
        pltpu.sync_copy(idx_hbm.at[pl.ds(base, _BPW)], idx_v)

        for v in range(_BPW // 16):
            rows = idx_v[pl.ds(v * 16, 16)]
            offs = rows * _DIM
            for dd in range(_DIM):
                vals = plsc.load_gather(table_v, [offs + dd])
                stage_v[dd, pl.ds(v * 16, 16)] = vals
        bb = wid // 2
        hh = wid % 2
        pltpu.sync_copy(stage_v, out_hbm.at[bb, :, pl.ds(hh * _HPX, _HPX)])

    return _sc_gather


def kernel(latents, embedding_weight):
    b, d, h, w = latents.shape
    zt3 = latents.reshape(b, d, h * w)
    idx, loss, perp = _tc_call(zt3, embedding_weight)
    q3 = _sc_gather_call()(embedding_weight.reshape(-1), idx)
    quantized = q3.reshape(b, d, h, w)
    return quantized, loss[0, 0], perp[0, 0]


# speedup vs baseline: 1.4364x; 1.4364x over previous
import functools

import jax
import jax.numpy as jnp
from jax import lax
from jax.experimental import pallas as pl
from jax.experimental.pallas import tpu as pltpu
from jax.experimental.pallas import tpu_sc as plsc

_NUM_EMB = 1024
_DIM = 64
_CC = 0.25
_ROWS = 16384
_TILE = 1024
_GRID = _ROWS // _TILE


_BB = 2
_GRID2 = _GRID // _BB


def _vq_tc_body(z_ref, e_ref, idx_ref, loss_ref, perp_ref, counts_ref,
                sse_ref):
    i = pl.program_id(0)
    e = e_ref[...]
    e2 = jnp.sum(e * e, axis=1)
    rows = lax.broadcasted_iota(jnp.int32, (_NUM_EMB, _TILE), 0)
    cbs = []
    sds = []
    for k in range(_BB):
        zt = z_ref[k]
        mmn = lax.dot_general(e, -2.0 * zt, (((1,), (0,)), ((), ())))
        z2 = jnp.sum(zt * zt, axis=0)
        s = (e2[:, None] + z2[None, :]) + mmn
        md = jnp.min(s, axis=0)
        idx = jnp.min(jnp.where(s == md[None, :], rows, _NUM_EMB), axis=0)
        idx_ref[pl.ds(k * _TILE, _TILE)] = idx
        onehot = (rows == idx[None, :]).astype(jnp.float32)
        cbs.append(lax.dot_general(onehot, jnp.ones((_TILE, 1), jnp.float32),
                                   (((1,), (0,)), ((), ()))))
        sds.append(jnp.sum(md))
    cb = sum(cbs)
    sd = sum(sds)

    @pl.when(i == 0)
    def _():
        counts_ref[...] = cb
        sse_ref[0] = sd

    @pl.when(i > 0)
    def _():
        counts_ref[...] = counts_ref[...] + cb
        sse_ref[0] = sse_ref[0] + sd

    @pl.when(i == _GRID2 - 1)
    def _():
        loss_ref[0, 0] = (1.0 + _CC) * sse_ref[0] / (_ROWS * _DIM)
        p = counts_ref[...] * (1.0 / _ROWS)
        ent = jnp.sum(p * jnp.log(p + 1e-10))
        perp_ref[0, 0] = jnp.exp(-ent)


def _tc_call(zt3, emb):
    return pl.pallas_call(
        _vq_tc_body,
        grid=(_GRID2,),
        in_specs=[
            pl.BlockSpec((_BB, _DIM, _TILE), lambda i: (i, 0, 0)),
            pl.BlockSpec((_NUM_EMB, _DIM), lambda i: (0, 0)),
        ],
        out_specs=[
            pl.BlockSpec((_BB * _TILE,), lambda i: (i,)),
            pl.BlockSpec(memory_space=pltpu.SMEM),
            pl.BlockSpec(memory_space=pltpu.SMEM),
        ],
        out_shape=[
            jax.ShapeDtypeStruct((_ROWS,), jnp.int32),
            jax.ShapeDtypeStruct((1, 1), jnp.float32),
            jax.ShapeDtypeStruct((1, 1), jnp.float32),
        ],
        scratch_shapes=[
            pltpu.VMEM((_NUM_EMB, 1), jnp.float32),
            pltpu.SMEM((1,), jnp.float32),
        ],
    )(zt3, emb)


_NC, _NS = 2, 16
_NW = _NC * _NS
_BPW = _ROWS // _NW
_NB = 16
_HPX = 512


_DPAD = 128


@functools.cache
def _sc_gather_call():
    @functools.partial(
        pl.kernel,
        mesh=plsc.VectorSubcoreMesh(core_axis_name="c", subcore_axis_name="s"),
        out_type=jax.ShapeDtypeStruct((_ROWS, _DPAD), jnp.float32),
        scratch_types=[
            pltpu.VMEM((_BPW,), jnp.int32),
            pltpu.VMEM((_BPW, _DPAD), jnp.float32),
            pltpu.SemaphoreType.DMA,
        ],
    )
    def _sc_gather(table_hbm, idx_hbm, out_hbm, idx_v, rows_v, sem):
        wid = lax.axis_index("s") * _NC + lax.axis_index("c")
        base = wid * _BPW
        pltpu.sync_copy(idx_hbm.at[pl.ds(base, _BPW)], idx_v)
        pltpu.async_copy(table_hbm.at[idx_v], rows_v, sem).wait()
        pltpu.sync_copy(rows_v, out_hbm.at[pl.ds(base, _BPW)])

    return _sc_gather


def kernel(latents, embedding_weight):
    b, d, h, w = latents.shape
    zt3 = latents.reshape(b, d, h * w)
    idx, loss, perp = _tc_call(zt3, embedding_weight)
    emb_pad = jnp.pad(embedding_weight, ((0, 0), (0, _DPAD - d)))
    qflat = _sc_gather_call()(emb_pad, idx)[:, :d]
    quantized = jnp.transpose(qflat.reshape(b, h, w, d), (0, 3, 1, 2))
    return quantized, loss[0, 0], perp[0, 0]


# speedup vs baseline: 1.4989x; 1.0434x over previous
import functools

import jax
import jax.numpy as jnp
from jax import lax
from jax.experimental import pallas as pl
from jax.experimental.pallas import tpu as pltpu
from jax.experimental.pallas import tpu_sc as plsc

_NUM_EMB = 1024
_DIM = 64
_CC = 0.25
_ROWS = 16384
_TILE = 1024
_GRID = _ROWS // _TILE


_BB = 2
_GRID2 = _GRID // _BB


def _vq_tc_body(z_ref, e_ref, idx_ref, loss_ref, perp_ref, oh_ref,
                sse_ref):
    i = pl.program_id(0)
    e = e_ref[...]
    e2 = jnp.sum(e * e, axis=1)
    rows_f = lax.broadcasted_iota(
        jnp.int32, (_NUM_EMB, _TILE), 0).astype(jnp.float32)
    ohs = []
    sds = []
    for k in range(_BB):
        zt = z_ref[k]
        mmn = lax.dot_general(-2.0 * e, zt, (((1,), (0,)), ((), ())))
        z2 = jnp.sum(zt * zt, axis=0)
        s = (e2[:, None] + z2[None, :]) + mmn
        md = jnp.min(s, axis=0)
        cand = jnp.where(s == md[None, :], rows_f, float(_NUM_EMB))
        idx_f = jnp.min(cand, axis=0)
        idx_ref[pl.ds(k * _TILE, _TILE)] = idx_f.astype(jnp.int32)
        ohs.append((cand == idx_f[None, :]).astype(jnp.float32))
        sds.append(jnp.sum(md))
    oh = sum(ohs)
    sd = sum(sds)

    @pl.when(i == 0)
    def _():
        oh_ref[...] = oh
        sse_ref[0] = sd

    @pl.when(i > 0)
    def _():
        oh_ref[...] = oh_ref[...] + oh
        sse_ref[0] = sse_ref[0] + sd

    @pl.when(i == _GRID2 - 1)
    def _():
        loss_ref[0, 0] = (1.0 + _CC) * sse_ref[0] / (_ROWS * _DIM)
        cb = lax.dot_general(oh_ref[...], jnp.ones((_TILE, 1), jnp.float32),
                             (((1,), (0,)), ((), ())))
        p = cb * (1.0 / _ROWS)
        ent = jnp.sum(p * jnp.log(p + 1e-10))
        perp_ref[0, 0] = jnp.exp(-ent)


def _tc_call(zt3, emb):
    return pl.pallas_call(
        _vq_tc_body,
        grid=(_GRID2,),
        in_specs=[
            pl.BlockSpec((_BB, _DIM, _TILE), lambda i: (i, 0, 0)),
            pl.BlockSpec((_NUM_EMB, _DIM), lambda i: (0, 0)),
        ],
        out_specs=[
            pl.BlockSpec((_BB * _TILE,), lambda i: (i,)),
            pl.BlockSpec(memory_space=pltpu.SMEM),
            pl.BlockSpec(memory_space=pltpu.SMEM),
        ],
        out_shape=[
            jax.ShapeDtypeStruct((_ROWS,), jnp.int32),
            jax.ShapeDtypeStruct((1, 1), jnp.float32),
            jax.ShapeDtypeStruct((1, 1), jnp.float32),
        ],
        scratch_shapes=[
            pltpu.VMEM((_NUM_EMB, _TILE), jnp.float32),
            pltpu.SMEM((1,), jnp.float32),
        ],
    )(zt3, emb)


_NC, _NS = 2, 16
_NW = _NC * _NS
_BPW = _ROWS // _NW
_NB = 16
_HPX = 512


_DPAD = 128


@functools.cache
def _sc_gather_call():
    @functools.partial(
        pl.kernel,
        mesh=plsc.VectorSubcoreMesh(core_axis_name="c", subcore_axis_name="s"),
        out_type=jax.ShapeDtypeStruct((_ROWS, _DPAD), jnp.float32),
        scratch_types=[
            pltpu.VMEM((_BPW,), jnp.int32),
            pltpu.VMEM((_BPW, _DPAD), jnp.float32),
            pltpu.SemaphoreType.DMA,
        ],
    )
    def _sc_gather(table_hbm, idx_hbm, out_hbm, idx_v, rows_v, sem):
        wid = lax.axis_index("s") * _NC + lax.axis_index("c")
        base = wid * _BPW
        pltpu.sync_copy(idx_hbm.at[pl.ds(base, _BPW)], idx_v)
        pltpu.async_copy(table_hbm.at[idx_v], rows_v, sem).wait()
        pltpu.sync_copy(rows_v, out_hbm.at[pl.ds(base, _BPW)])

    return _sc_gather


def kernel(latents, embedding_weight):
    b, d, h, w = latents.shape
    zt3 = latents.reshape(b, d, h * w)
    idx, loss, perp = _tc_call(zt3, embedding_weight)
    emb_pad = jnp.pad(embedding_weight, ((0, 0), (0, _DPAD - d)))
    qflat = _sc_gather_call()(emb_pad, idx)[:, :d]
    quantized = jnp.transpose(qflat.reshape(b, h, w, d), (0, 3, 1, 2))
    return quantized, loss[0, 0], perp[0, 0]


# speedup vs baseline: 1.5336x; 1.0232x over previous
import functools

import jax
import jax.numpy as jnp
from jax import lax
from jax.experimental import pallas as pl
from jax.experimental.pallas import tpu as pltpu
from jax.experimental.pallas import tpu_sc as plsc

_NUM_EMB = 1024
_DIM = 64
_CC = 0.25
_ROWS = 16384
_TILE = 1024
_GRID = _ROWS // _TILE


_BB = 4
_GRID2 = _GRID // _BB


def _vq_tc_body(z_ref, e_ref, idx_ref, loss_ref, perp_ref, oh_ref,
                sse_ref):
    i = pl.program_id(0)
    e = e_ref[...]
    e2 = jnp.sum(e * e, axis=1)
    rows_f = lax.broadcasted_iota(
        jnp.int32, (_NUM_EMB, _TILE), 0).astype(jnp.float32)
    ohs = []
    sds = []
    for k in range(_BB):
        zt = z_ref[k]
        mmn = lax.dot_general(-2.0 * e, zt, (((1,), (0,)), ((), ())))
        z2 = jnp.sum(zt * zt, axis=0)
        s = (e2[:, None] + z2[None, :]) + mmn
        md = jnp.min(s, axis=0)
        cand = jnp.where(s == md[None, :], rows_f, float(_NUM_EMB))
        idx_f = jnp.min(cand, axis=0)
        idx_ref[pl.ds(k * _TILE, _TILE)] = idx_f.astype(jnp.int32)
        ohs.append((cand == idx_f[None, :]).astype(jnp.float32))
        sds.append(jnp.sum(md))
    oh = sum(ohs)
    sd = sum(sds)

    @pl.when(i == 0)
    def _():
        oh_ref[...] = oh
        sse_ref[0] = sd

    @pl.when(i > 0)
    def _():
        oh_ref[...] = oh_ref[...] + oh
        sse_ref[0] = sse_ref[0] + sd

    @pl.when(i == _GRID2 - 1)
    def _():
        loss_ref[0, 0] = (1.0 + _CC) * sse_ref[0] / (_ROWS * _DIM)
        cb = lax.dot_general(oh_ref[...], jnp.ones((_TILE, 1), jnp.float32),
                             (((1,), (0,)), ((), ())))
        p = cb * (1.0 / _ROWS)
        ent = jnp.sum(p * jnp.log(p + 1e-10))
        perp_ref[0, 0] = jnp.exp(-ent)


def _tc_call(zt3, emb):
    return pl.pallas_call(
        _vq_tc_body,
        grid=(_GRID2,),
        in_specs=[
            pl.BlockSpec((_BB, _DIM, _TILE), lambda i: (i, 0, 0)),
            pl.BlockSpec((_NUM_EMB, _DIM), lambda i: (0, 0)),
        ],
        out_specs=[
            pl.BlockSpec((_BB * _TILE,), lambda i: (i,)),
            pl.BlockSpec(memory_space=pltpu.SMEM),
            pl.BlockSpec(memory_space=pltpu.SMEM),
        ],
        out_shape=[
            jax.ShapeDtypeStruct((_ROWS,), jnp.int32),
            jax.ShapeDtypeStruct((1, 1), jnp.float32),
            jax.ShapeDtypeStruct((1, 1), jnp.float32),
        ],
        scratch_shapes=[
            pltpu.VMEM((_NUM_EMB, _TILE), jnp.float32),
            pltpu.SMEM((1,), jnp.float32),
        ],
    )(zt3, emb)


_NC, _NS = 2, 16
_NW = _NC * _NS
_BPW = _ROWS // _NW
_NB = 16
_HPX = 512


_DPAD = 128


@functools.cache
def _sc_gather_call():
    @functools.partial(
        pl.kernel,
        mesh=plsc.VectorSubcoreMesh(core_axis_name="c", subcore_axis_name="s"),
        out_type=jax.ShapeDtypeStruct((_ROWS, _DPAD), jnp.float32),
        scratch_types=[
            pltpu.VMEM((_BPW,), jnp.int32),
            pltpu.VMEM((_BPW, _DPAD), jnp.float32),
            pltpu.SemaphoreType.DMA,
        ],
    )
    def _sc_gather(table_hbm, idx_hbm, out_hbm, idx_v, rows_v, sem):
        wid = lax.axis_index("s") * _NC + lax.axis_index("c")
        base = wid * _BPW
        pltpu.sync_copy(idx_hbm.at[pl.ds(base, _BPW)], idx_v)
        pltpu.async_copy(table_hbm.at[idx_v], rows_v, sem).wait()
        pltpu.sync_copy(rows_v, out_hbm.at[pl.ds(base, _BPW)])

    return _sc_gather


def kernel(latents, embedding_weight):
    b, d, h, w = latents.shape
    zt3 = latents.reshape(b, d, h * w)
    idx, loss, perp = _tc_call(zt3, embedding_weight)
    emb_pad = jnp.pad(embedding_weight, ((0, 0), (0, _DPAD - d)))
    qflat = _sc_gather_call()(emb_pad, idx)[:, :d]
    quantized = jnp.transpose(qflat.reshape(b, h, w, d), (0, 3, 1, 2))
    return quantized, loss[0, 0], perp[0, 0]
